# Initial kernel scaffold; baseline (speedup 1.0000x reference)
import jax
import jax.numpy as jnp
from jax.experimental import pallas as pl


def kernel(E, V, edge_index, W1, b1, W2, b2):
    raise NotImplementedError("write your pallas kernel here")


# speedup vs baseline: 3.5546x; 3.5546x over previous
import functools

import jax
import jax.numpy as jnp
from jax import lax
from jax.experimental import pallas as pl
from jax.experimental.pallas import tpu as pltpu
from jax.experimental.pallas import tpu_sc as plsc

EDGE_DIM = 16
NODE_DIM = 128
HID = 128
N_NODES = 10000
N_EDGES = 320000

NC = 2
NS = 16
NW = NC * NS

CHUNK = 128
N_CHUNKS = N_EDGES // CHUNK
MAXC = -(-N_CHUNKS // NW)
LANES = 16


def _pre_body(v_ref, ws_ref, wd_ref, us_ref, ud_ref):
    v = v_ref[...]
    us_ref[...] = jnp.dot(v, ws_ref[...], preferred_element_type=jnp.float32)
    ud_ref[...] = jnp.dot(v, wd_ref[...], preferred_element_type=jnp.float32)


def _precompute_tables(V, W1s, W1d):
    return pl.pallas_call(
        _pre_body,
        out_shape=(
            jax.ShapeDtypeStruct((N_NODES, HID), jnp.float32),
            jax.ShapeDtypeStruct((N_NODES, HID), jnp.float32),
        ),
    )(V, W1s, W1d)


def _sc_body(us_hbm, ud_hbm, src_hbm, dst_hbm, out_hbm,
             idx_s, idx_d, rows_s, rows_d, sem):
    cid = lax.axis_index("c")
    sid = lax.axis_index("s")
    wid = sid * NC + cid
    lo = (wid * N_CHUNKS) // NW
    hi = ((wid + 1) * N_CHUNKS) // NW
    n = hi - lo
    base = pl.multiple_of(lo * CHUNK, CHUNK)
    pltpu.sync_copy(src_hbm.at[pl.ds(base, MAXC * CHUNK)], idx_s)
    pltpu.sync_copy(dst_hbm.at[pl.ds(base, MAXC * CHUNK)], idx_d)

    def chunk_body(i, carry):
        off = i * CHUNK
        c1 = pltpu.async_copy(us_hbm.at[idx_s.at[pl.ds(off, CHUNK)]], rows_s, sem)
        c2 = pltpu.async_copy(ud_hbm.at[idx_d.at[pl.ds(off, CHUNK)]], rows_d, sem)
        c1.wait()
        c2.wait()

        def add_row(r, carry2):
            for j in range(HID // LANES):
                sl = pl.ds(j * LANES, LANES)
                rows_s[r, sl] = rows_s[r, sl] + rows_d[r, sl]
            return carry2

        lax.fori_loop(0, CHUNK, add_row, 0)
        pltpu.sync_copy(rows_s, out_hbm.at[pl.ds(base + off, CHUNK), :])
        return carry

    lax.fori_loop(0, n, chunk_body, 0)


_sc_gather_add = functools.partial(
    pl.kernel,
    mesh=plsc.VectorSubcoreMesh(core_axis_name="c", subcore_axis_name="s"),
    out_type=jax.ShapeDtypeStruct((N_EDGES, HID), jnp.float32),
    scratch_types=[
        pltpu.VMEM((MAXC * CHUNK,), jnp.int32),
        pltpu.VMEM((MAXC * CHUNK,), jnp.int32),
        pltpu.VMEM((CHUNK, HID), jnp.float32),
        pltpu.VMEM((CHUNK, HID), jnp.float32),
        pltpu.SemaphoreType.DMA,
    ],
)(_sc_body)


BLK = 3200


def _mlp_body(e_ref, g_ref, w1e_ref, b1_ref, w2_ref, b2_ref, out_ref):
    h = jnp.dot(e_ref[...], w1e_ref[...], preferred_element_type=jnp.float32)
    h = h + g_ref[...] + b1_ref[...]
    h = jnp.maximum(h, 0.0)
    out_ref[...] = (
        jnp.dot(h, w2_ref[...], preferred_element_type=jnp.float32) + b2_ref[...]
    )


def _mlp(E, G, W1e, b1, W2, b2):
    grid = N_EDGES // BLK
    return pl.pallas_call(
        _mlp_body,
        grid=(grid,),
        in_specs=[
            pl.BlockSpec((BLK, EDGE_DIM), lambda i: (i, 0)),
            pl.BlockSpec((BLK, HID), lambda i: (i, 0)),
            pl.BlockSpec((EDGE_DIM, HID), lambda i: (0, 0)),
            pl.BlockSpec((1, HID), lambda i: (0, 0)),
            pl.BlockSpec((HID, HID), lambda i: (0, 0)),
            pl.BlockSpec((1, HID), lambda i: (0, 0)),
        ],
        out_specs=pl.BlockSpec((BLK, HID), lambda i: (i, 0)),
        out_shape=jax.ShapeDtypeStruct((N_EDGES, HID), jnp.float32),
    )(E, G, W1e, b1, W2, b2)


def kernel(E, V, edge_index, W1, b1, W2, b2):
    src = edge_index[0].astype(jnp.int32)
    dst = edge_index[1].astype(jnp.int32)
    W1e = W1[:EDGE_DIM]
    W1s = W1[EDGE_DIM:EDGE_DIM + NODE_DIM]
    W1d = W1[EDGE_DIM + NODE_DIM:]
    U_s, U_d = _precompute_tables(V, W1s, W1d)
    G = _sc_gather_add(U_s, U_d, src, dst)
    return _mlp(E, G, W1e, b1.reshape(1, HID), W2, b2.reshape(1, HID))


# speedup vs baseline: 4.3360x; 1.2198x over previous
import functools

import jax
import jax.numpy as jnp
from jax import lax
from jax.experimental import pallas as pl
from jax.experimental.pallas import tpu as pltpu
from jax.experimental.pallas import tpu_sc as plsc

EDGE_DIM = 16
NODE_DIM = 128
HID = 128
N_NODES = 10000
N_EDGES = 320000

NC = 2
NS = 16
NW = NC * NS

CHUNK = 80
EDGES_PER_TILE = N_EDGES // NW
TILE_CHUNKS = EDGES_PER_TILE // CHUNK
NSLOT = 4
LANES = 16


def _pre_body(v_ref, ws_ref, wd_ref, us_ref, ud_ref):
    v = v_ref[...]
    us_ref[...] = jnp.dot(v, ws_ref[...], preferred_element_type=jnp.float32)
    ud_ref[...] = jnp.dot(v, wd_ref[...], preferred_element_type=jnp.float32)


def _precompute_tables(V, W1s, W1d):
    return pl.pallas_call(
        _pre_body,
        out_shape=(
            jax.ShapeDtypeStruct((N_NODES, HID), jnp.float32),
            jax.ShapeDtypeStruct((N_NODES, HID), jnp.float32),
        ),
    )(V, W1s, W1d)


def _sc_body(us_hbm, ud_hbm, src_hbm, dst_hbm, out_hbm,
             idx_s, idx_d, rows_s, rows_d, gsems, wsems):
    cid = lax.axis_index("c")
    sid = lax.axis_index("s")
    wid = sid * NC + cid
    base = pl.multiple_of(wid * EDGES_PER_TILE, CHUNK)
    n = TILE_CHUNKS
    pltpu.sync_copy(src_hbm.at[pl.ds(base, EDGES_PER_TILE)], idx_s)
    pltpu.sync_copy(dst_hbm.at[pl.ds(base, EDGES_PER_TILE)], idx_d)

    def issue_gather(b, i):
        off = pl.multiple_of(i * CHUNK, CHUNK)
        pltpu.make_async_copy(
            us_hbm.at[idx_s.at[pl.ds(off, CHUNK)]], rows_s[b], gsems[b]).start()
        pltpu.make_async_copy(
            ud_hbm.at[idx_d.at[pl.ds(off, CHUNK)]], rows_d[b], gsems[b]).start()

    def wait_gather(b, i):
        off = pl.multiple_of(i * CHUNK, CHUNK)
        pltpu.make_async_copy(
            us_hbm.at[idx_s.at[pl.ds(off, CHUNK)]], rows_s[b], gsems[b]).wait()
        pltpu.make_async_copy(
            ud_hbm.at[idx_d.at[pl.ds(off, CHUNK)]], rows_d[b], gsems[b]).wait()

    def issue_write(b, i):
        off = pl.multiple_of(base + i * CHUNK, CHUNK)
        pltpu.make_async_copy(
            rows_s[b], out_hbm.at[pl.ds(off, CHUNK), :], wsems[b]).start()

    def wait_write(b):
        pltpu.make_async_copy(
            rows_s[b], out_hbm.at[pl.ds(0, CHUNK), :], wsems[b]).wait()

    issue_gather(0, 0)
    issue_gather(1, 1)

    def quad(j, carry):
        for b in range(NSLOT):
            i = j * NSLOT + b
            bn = (b + 2) % NSLOT

            @pl.when((i >= 2) & (i < n))
            def _():
                wait_write(bn)

            @pl.when(i + 2 < n)
            def _():
                issue_gather(bn, i + 2)

            @pl.when(i < n)
            def _():
                wait_gather(b, i)
                rs = rows_s[b]
                rd = rows_d[b]

                @plsc.parallel_loop(0, CHUNK, unroll=4)
                def _add(r):
                    for jj in range(HID // LANES):
                        sl = pl.ds(jj * LANES, LANES)
                        rs[r, sl] = rs[r, sl] + rd[r, sl]

                issue_write(b, i)
        return carry

    lax.fori_loop(0, (n + NSLOT - 1) // NSLOT, quad, 0)
    wait_write((n - 2) % NSLOT)
    wait_write((n - 1) % NSLOT)


_sc_gather_add = functools.partial(
    pl.kernel,
    mesh=plsc.VectorSubcoreMesh(core_axis_name="c", subcore_axis_name="s"),
    out_type=jax.ShapeDtypeStruct((N_EDGES, HID), jnp.float32),
    scratch_types=[
        pltpu.VMEM((EDGES_PER_TILE,), jnp.int32),
        pltpu.VMEM((EDGES_PER_TILE,), jnp.int32),
        [pltpu.VMEM((CHUNK, HID), jnp.float32) for _ in range(NSLOT)],
        [pltpu.VMEM((CHUNK, HID), jnp.float32) for _ in range(NSLOT)],
        [pltpu.SemaphoreType.DMA for _ in range(NSLOT)],
        [pltpu.SemaphoreType.DMA for _ in range(NSLOT)],
    ],
)(_sc_body)


BLK = 3200


def _mlp_body(e_ref, g_ref, w1e_ref, b1_ref, w2_ref, b2_ref, out_ref):
    h = jnp.dot(e_ref[...], w1e_ref[...], preferred_element_type=jnp.float32)
    h = h + g_ref[...] + b1_ref[...]
    h = jnp.maximum(h, 0.0)
    out_ref[...] = (
        jnp.dot(h, w2_ref[...], preferred_element_type=jnp.float32) + b2_ref[...]
    )


def _mlp(E, G, W1e, b1, W2, b2):
    grid = N_EDGES // BLK
    return pl.pallas_call(
        _mlp_body,
        grid=(grid,),
        in_specs=[
            pl.BlockSpec((BLK, EDGE_DIM), lambda i: (i, 0)),
            pl.BlockSpec((BLK, HID), lambda i: (i, 0)),
            pl.BlockSpec((EDGE_DIM, HID), lambda i: (0, 0)),
            pl.BlockSpec((1, HID), lambda i: (0, 0)),
            pl.BlockSpec((HID, HID), lambda i: (0, 0)),
            pl.BlockSpec((1, HID), lambda i: (0, 0)),
        ],
        out_specs=pl.BlockSpec((BLK, HID), lambda i: (i, 0)),
        out_shape=jax.ShapeDtypeStruct((N_EDGES, HID), jnp.float32),
    )(E, G, W1e, b1, W2, b2)


def kernel(E, V, edge_index, W1, b1, W2, b2):
    src = edge_index[0].astype(jnp.int32)
    dst = edge_index[1].astype(jnp.int32)
    W1e = W1[:EDGE_DIM]
    W1s = W1[EDGE_DIM:EDGE_DIM + NODE_DIM]
    W1d = W1[EDGE_DIM + NODE_DIM:]
    U_s, U_d = _precompute_tables(V, W1s, W1d)
    G = _sc_gather_add(U_s, U_d, src, dst)
    return _mlp(E, G, W1e, b1.reshape(1, HID), W2, b2.reshape(1, HID))
